# baseline (speedup 1.0000x reference)
import jax
import jax.numpy as jnp
from jax.experimental import pallas as pl
from jax.experimental.pallas import tpu as pltpu

_BT = 512
_K = 8


def _router_block(h_ref, w_ref, mask_ref, probs_ref, logits_ref, logits2_ref):
    logits = jax.lax.dot_general(
        h_ref[...], w_ref[...], (((1,), (1,)), ((), ())),
        preferred_element_type=jnp.float32,
        precision=jax.lax.Precision.DEFAULT,
    )
    logits_ref[...] = logits
    logits2_ref[...] = logits

    m = jnp.max(logits, axis=1, keepdims=True)
    ex = jnp.exp(logits - m)
    probs_ref[...] = ex / jnp.sum(ex, axis=1, keepdims=True)

    x = logits
    for _ in range(_K):
        mx = jnp.max(x, axis=1, keepdims=True)
        x = jnp.where(x == mx, -jnp.inf, x)
    mask_ref[...] = jnp.isneginf(x)


@jax.jit
def kernel(h, W):
    t, d = h.shape
    e = W.shape[0]
    mask, probs, logits, logits2 = pl.pallas_call(
        _router_block,
        grid=(t // _BT,),
        in_specs=[
            pl.BlockSpec((_BT, d), lambda i: (i, 0)),
            pl.BlockSpec((e, d), lambda i: (0, 0)),
        ],
        out_specs=[
            pl.BlockSpec((_BT, e), lambda i: (i, 0)),
            pl.BlockSpec((_BT, e), lambda i: (i, 0)),
            pl.BlockSpec((_BT, e), lambda i: (i, 0)),
            pl.BlockSpec((_BT, e), lambda i: (i, 0)),
        ],
        out_shape=[
            jax.ShapeDtypeStruct((t, e), jnp.bool_),
            jax.ShapeDtypeStruct((t, e), jnp.float32),
            jax.ShapeDtypeStruct((t, e), jnp.float32),
            jax.ShapeDtypeStruct((t, e), jnp.float32),
        ],
        compiler_params=pltpu.CompilerParams(
            dimension_semantics=("parallel",),
        ),
    )(h, W)
    return (mask, probs, logits, logits2)


# speedup vs baseline: 1.3578x; 1.3578x over previous
import jax
import jax.numpy as jnp
from jax.experimental import pallas as pl
from jax.experimental.pallas import tpu as pltpu

_BT = 2048
_BD = 2048
_K = 8


def _router_block(h_ref, w_ref, mask_ref, probs_ref, logits_ref, logits2_ref,
                  acc_ref):
    j = pl.program_id(1)
    part = jax.lax.dot_general(
        w_ref[...], h_ref[...], (((1,), (1,)), ((), ())),
        preferred_element_type=jnp.float32,
        precision=jax.lax.Precision.DEFAULT,
    )

    @pl.when(j == 0)
    def _():
        acc_ref[...] = part

    @pl.when(j != 0)
    def _():
        acc_ref[...] += part

    @pl.when(j == pl.num_programs(1) - 1)
    def _():
        logits = acc_ref[...]
        logits_ref[...] = logits
        logits2_ref[...] = logits

        m = jnp.max(logits, axis=0, keepdims=True)
        ex = jnp.exp(logits - m)
        probs_ref[...] = ex / jnp.sum(ex, axis=0, keepdims=True)

        x = logits
        for _ in range(_K):
            mx = jnp.max(x, axis=0, keepdims=True)
            x = jnp.where(x == mx, -jnp.inf, x)
        mask_ref[...] = jnp.isneginf(x).astype(jnp.int8)


@jax.jit
def kernel(h, W):
    t, d = h.shape
    e = W.shape[0]
    mask_t, probs_t, logits_t, logits2_t = pl.pallas_call(
        _router_block,
        grid=(t // _BT, d // _BD),
        in_specs=[
            pl.BlockSpec((_BT, _BD), lambda i, j: (i, j)),
            pl.BlockSpec((e, _BD), lambda i, j: (0, j)),
        ],
        out_specs=[
            pl.BlockSpec((e, _BT), lambda i, j: (0, i)),
            pl.BlockSpec((e, _BT), lambda i, j: (0, i)),
            pl.BlockSpec((e, _BT), lambda i, j: (0, i)),
            pl.BlockSpec((e, _BT), lambda i, j: (0, i)),
        ],
        out_shape=[
            jax.ShapeDtypeStruct((e, t), jnp.int8),
            jax.ShapeDtypeStruct((e, t), jnp.float32),
            jax.ShapeDtypeStruct((e, t), jnp.float32),
            jax.ShapeDtypeStruct((e, t), jnp.float32),
        ],
        scratch_shapes=[pltpu.VMEM((e, _BT), jnp.float32)],
        compiler_params=pltpu.CompilerParams(
            dimension_semantics=("parallel", "arbitrary"),
        ),
    )(h, W)
    mask = mask_t.T.astype(jnp.bool_)
    return (mask, probs_t.T, logits_t.T, logits2_t.T)


# speedup vs baseline: 1.4044x; 1.0343x over previous
import jax
import jax.numpy as jnp
from jax.experimental import pallas as pl
from jax.experimental.pallas import tpu as pltpu

_BT = 1024
_BH = _BT // 2
_K = 8


def _router_block(h1_ref, h2_ref, w_ref, mask_ref, probs_ref, logits_ref,
                  logits2_ref):
    w = w_ref[...]
    for half, h_ref in enumerate((h1_ref, h2_ref)):
        sl = pl.ds(half * _BH, _BH)
        logits = jax.lax.dot_general(
            w, h_ref[...], (((1,), (1,)), ((), ())),
            preferred_element_type=jnp.float32,
            precision=jax.lax.Precision.DEFAULT,
        )
        logits_ref[:, sl] = logits
        logits2_ref[:, sl] = logits

        m = jnp.max(logits, axis=0, keepdims=True)
        ex = jnp.exp(logits - m)
        probs_ref[:, sl] = ex / jnp.sum(ex, axis=0, keepdims=True)

        x = logits
        for _ in range(_K):
            mx = jnp.max(x, axis=0, keepdims=True)
            x = jnp.where(x == mx, -jnp.inf, x)
        mask_ref[:, sl] = jnp.isneginf(x).astype(jnp.int8)


@jax.jit
def kernel(h, W):
    t, d = h.shape
    e = W.shape[0]
    mask_t, probs_t, logits_t, logits2_t = pl.pallas_call(
        _router_block,
        grid=(t // _BT,),
        in_specs=[
            pl.BlockSpec((_BH, d), lambda i: (2 * i, 0)),
            pl.BlockSpec((_BH, d), lambda i: (2 * i + 1, 0)),
            pl.BlockSpec((e, d), lambda i: (0, 0)),
        ],
        out_specs=[
            pl.BlockSpec((e, _BT), lambda i: (0, i)),
            pl.BlockSpec((e, _BT), lambda i: (0, i)),
            pl.BlockSpec((e, _BT), lambda i: (0, i)),
            pl.BlockSpec((e, _BT), lambda i: (0, i)),
        ],
        out_shape=[
            jax.ShapeDtypeStruct((e, t), jnp.int8),
            jax.ShapeDtypeStruct((e, t), jnp.float32),
            jax.ShapeDtypeStruct((e, t), jnp.float32),
            jax.ShapeDtypeStruct((e, t), jnp.float32),
        ],
        compiler_params=pltpu.CompilerParams(
            dimension_semantics=("parallel",),
        ),
    )(h, h, W)
    mask = mask_t.T.astype(jnp.bool_)
    return (mask, probs_t.T, logits_t.T, logits2_t.T)


# speedup vs baseline: 1.4056x; 1.0009x over previous
import jax
import jax.numpy as jnp
from jax.experimental import pallas as pl
from jax.experimental.pallas import tpu as pltpu

_BT = 1024
_K = 8


def _router_block(h_ref, w_ref, mask_ref, probs_ref, logits_ref, logits2_ref):
    logits = jax.lax.dot_general(
        w_ref[...], h_ref[...], (((1,), (1,)), ((), ())),
        preferred_element_type=jnp.float32,
        precision=jax.lax.Precision.DEFAULT,
    )
    logits_ref[...] = logits
    logits2_ref[...] = logits

    m = jnp.max(logits, axis=0, keepdims=True)
    ex = jnp.exp(logits - m)
    probs_ref[...] = ex / jnp.sum(ex, axis=0, keepdims=True)

    x = logits
    for _ in range(_K):
        mx = jnp.max(x, axis=0, keepdims=True)
        x = jnp.where(x == mx, -jnp.inf, x)
    mask_ref[...] = jnp.isneginf(x).astype(jnp.int8)


@jax.jit
def kernel(h, W):
    t, d = h.shape
    e = W.shape[0]
    mask_t, probs_t, logits_t, logits2_t = pl.pallas_call(
        _router_block,
        grid=(t // _BT,),
        in_specs=[
            pl.BlockSpec((_BT, d), lambda i: (i, 0)),
            pl.BlockSpec((e, d), lambda i: (0, 0)),
        ],
        out_specs=[
            pl.BlockSpec((e, _BT), lambda i: (0, i)),
            pl.BlockSpec((e, _BT), lambda i: (0, i)),
            pl.BlockSpec((e, _BT), lambda i: (0, i)),
            pl.BlockSpec((e, _BT), lambda i: (0, i)),
        ],
        out_shape=[
            jax.ShapeDtypeStruct((e, t), jnp.int8),
            jax.ShapeDtypeStruct((e, t), jnp.float32),
            jax.ShapeDtypeStruct((e, t), jnp.float32),
            jax.ShapeDtypeStruct((e, t), jnp.float32),
        ],
        compiler_params=pltpu.CompilerParams(
            dimension_semantics=("parallel",),
        ),
    )(h, W)
    mask = mask_t.T.astype(jnp.bool_)
    return (mask, probs_t.T, logits_t.T, logits2_t.T)
